# Initial kernel scaffold; baseline (speedup 1.0000x reference)
import jax
import jax.numpy as jnp
from jax.experimental import pallas as pl


def kernel(output):
    raise NotImplementedError("write your pallas kernel here")


# speedup vs baseline: 1.0250x; 1.0250x over previous
import functools

import jax
import jax.numpy as jnp
from jax.experimental import pallas as pl
from jax.experimental.pallas import tpu as pltpu


def _diag_loss_kernel(x_hbm, out_ref, buf, sems, *, batch, scale):
    def slice_copy(b):
        return pltpu.make_async_copy(x_hbm.at[b, b], buf.at[b], sems.at[b])

    for b in range(batch):
        slice_copy(b).start()
    for b in range(batch):
        slice_copy(b).wait()

    out_ref[0, 0] = jnp.sum(buf[...].astype(jnp.float32)) * jnp.float32(scale)


def kernel(output):
    B, C, H, W = output.shape
    hw = H * W
    scale = -1.0 / float(hw)

    if hw % 128 == 0:
        R, L = hw // 128, 128
    else:
        R, L = 1, hw
    x = output.reshape(B, C, R, L)

    loss = pl.pallas_call(
        functools.partial(_diag_loss_kernel, batch=B, scale=scale),
        out_shape=jax.ShapeDtypeStruct((1, 1), jnp.float32),
        in_specs=[pl.BlockSpec(memory_space=pl.ANY)],
        out_specs=pl.BlockSpec(memory_space=pltpu.SMEM),
        scratch_shapes=[
            pltpu.VMEM((B, R, L), output.dtype),
            pltpu.SemaphoreType.DMA((B,)),
        ],
    )(x)
    return loss[0, 0]
